# baseline (speedup 1.0000x reference)
import functools

import jax
import jax.numpy as jnp
from jax import lax
from jax.experimental import pallas as pl
from jax.experimental.pallas import tpu as pltpu
from jax.experimental.pallas import tpu_sc as plsc

LANES = 16
NC, NS = 2, 16
NW = NC * NS
CHUNK = 128


def kernel(head, tail, emb, W, b):
    B = head.shape[0]
    V, D = emb.shape
    bpw = B // NW
    n_chunks = bpw // CHUNK

    embf = emb.reshape(V * D)
    head3 = head.reshape(NW, n_chunks, CHUNK)
    tail3 = tail.reshape(NW, n_chunks, CHUNK)
    wb = jnp.concatenate([W.reshape(4), b, jnp.zeros((3,), jnp.float32)])
    wb16 = jnp.broadcast_to(wb[:, None], (8, LANES))

    @functools.partial(
        pl.kernel,
        out_type=jax.ShapeDtypeStruct((B,), jnp.float32),
        mesh=plsc.VectorSubcoreMesh(core_axis_name="c", subcore_axis_name="s"),
        scratch_types=[
            pltpu.VMEM((n_chunks, CHUNK), jnp.int32),
            pltpu.VMEM((n_chunks, CHUNK), jnp.int32),
            pltpu.VMEM((n_chunks, CHUNK), jnp.int32),
            pltpu.VMEM((n_chunks, CHUNK), jnp.int32),
            pltpu.VMEM((n_chunks, CHUNK), jnp.int32),
            pltpu.VMEM((n_chunks, CHUNK), jnp.int32),
            pltpu.VMEM((bpw,), jnp.float32),
            pltpu.VMEM((bpw,), jnp.float32),
            pltpu.VMEM((bpw,), jnp.float32),
            pltpu.VMEM((bpw,), jnp.float32),
            pltpu.VMEM((bpw,), jnp.float32),
            pltpu.VMEM((8, LANES), jnp.float32),
            pltpu.SemaphoreType.DMA,
        ],
    )
    def retrofit(head_h, tail_h, emb_h, wb_h, out_h,
                 hraw, traw, h0i, h1i, t0i, t1i,
                 h0v, h1v, t0v, t1v, outv, wbv, sem):
        wid = lax.axis_index("s") * NC + lax.axis_index("c")
        pltpu.sync_copy(head_h.at[wid], hraw)
        pltpu.sync_copy(tail_h.at[wid], traw)
        pltpu.sync_copy(wb_h, wbv)
        for c in range(n_chunks):
            for k in range(CHUNK // LANES):
                sl = (c, pl.ds(k * LANES, LANES))
                hd = hraw[sl] * D
                td = traw[sl] * D
                h0i[sl] = hd
                h1i[sl] = hd + 1
                t0i[sl] = td
                t1i[sl] = td + 1
        copies = []
        for c in range(n_chunks):
            dst = pl.ds(c * CHUNK, CHUNK)
            copies.append(pltpu.async_copy(emb_h.at[h0i.at[c]], h0v.at[dst], sem))
            copies.append(pltpu.async_copy(emb_h.at[h1i.at[c]], h1v.at[dst], sem))
            copies.append(pltpu.async_copy(emb_h.at[t0i.at[c]], t0v.at[dst], sem))
            copies.append(pltpu.async_copy(emb_h.at[t1i.at[c]], t1v.at[dst], sem))
        for cp in copies:
            cp.wait()
        w0, w1, w2, w3, bb = wbv[0], wbv[1], wbv[2], wbv[3], wbv[4]
        for k in range(bpw // LANES):
            sl = pl.ds(k * LANES, LANES)
            outv[sl] = (h0v[sl] * w0 + h1v[sl] * w1
                        + t0v[sl] * w2 + t1v[sl] * w3 + bb)
        pltpu.sync_copy(outv, out_h.at[pl.ds(wid * bpw, bpw)])

    return retrofit(head3, tail3, embf, wb16)


# speedup vs baseline: 3.3358x; 3.3358x over previous
import functools

import jax
import jax.numpy as jnp
from jax import lax
from jax.experimental import pallas as pl
from jax.experimental.pallas import tpu as pltpu
from jax.experimental.pallas import tpu_sc as plsc

LANES = 16
NC, NS = 2, 16
NW = NC * NS
CHUNK = 128


def kernel(head, tail, emb, W, b):
    B = head.shape[0]
    V, D = emb.shape
    bpw = B // NW
    n_chunks = bpw // CHUNK

    bits0 = lax.bitcast_convert_type(
        emb[:, 0].astype(jnp.bfloat16), jnp.uint16).astype(jnp.uint32)
    bits1 = lax.bitcast_convert_type(
        emb[:, 1].astype(jnp.bfloat16), jnp.uint16).astype(jnp.uint32)
    packed = ((bits0 << 16) | bits1).astype(jnp.int32)

    head3 = head.reshape(NW, n_chunks, CHUNK)
    tail3 = tail.reshape(NW, n_chunks, CHUNK)
    wb = jnp.concatenate([W.reshape(4), b, jnp.zeros((3,), jnp.float32)])
    wb16 = jnp.broadcast_to(wb[:, None], (8, LANES))

    @functools.partial(
        pl.kernel,
        out_type=jax.ShapeDtypeStruct((B,), jnp.float32),
        mesh=plsc.VectorSubcoreMesh(core_axis_name="c", subcore_axis_name="s"),
        compiler_params=pltpu.CompilerParams(needs_layout_passes=False),
        scratch_types=[
            pltpu.VMEM((n_chunks, CHUNK), jnp.int32),
            pltpu.VMEM((n_chunks, CHUNK), jnp.int32),
            pltpu.VMEM((bpw,), jnp.int32),
            pltpu.VMEM((bpw,), jnp.int32),
            pltpu.VMEM((bpw,), jnp.float32),
            pltpu.VMEM((8, LANES), jnp.float32),
            pltpu.SemaphoreType.DMA,
        ],
    )
    def retrofit(head_h, tail_h, packed_h, wb_h, out_h,
                 hidx, tidx, hw, tw, outv, wbv, sem):
        wid = lax.axis_index("s") * NC + lax.axis_index("c")
        pltpu.sync_copy(head_h.at[wid], hidx)
        pltpu.sync_copy(tail_h.at[wid], tidx)
        pltpu.sync_copy(wb_h, wbv)
        copies = []
        for c in range(n_chunks):
            dst = pl.ds(c * CHUNK, CHUNK)
            copies.append(pltpu.async_copy(packed_h.at[hidx.at[c]], hw.at[dst], sem))
            copies.append(pltpu.async_copy(packed_h.at[tidx.at[c]], tw.at[dst], sem))
        for cp in copies:
            cp.wait()
        w0, w1, w2, w3, bb = wbv[0], wbv[1], wbv[2], wbv[3], wbv[4]
        himask = jnp.full((LANES,), jnp.int32(-65536))

        def unpack2(g):
            hi = plsc.bitcast(g & himask, jnp.float32)
            lo = plsc.bitcast(g << 16, jnp.float32)
            return hi, lo

        for k in range(bpw // LANES):
            sl = pl.ds(k * LANES, LANES)
            h0, h1 = unpack2(hw[sl])
            t0, t1 = unpack2(tw[sl])
            outv[sl] = h0 * w0 + h1 * w1 + t0 * w2 + t1 * w3 + bb
        pltpu.sync_copy(outv, out_h.at[pl.ds(wid * bpw, bpw)])

    return retrofit(head3, tail3, packed, wb16)


# speedup vs baseline: 8.9367x; 2.6790x over previous
import functools

import jax
import jax.numpy as jnp
from jax import lax
from jax.experimental import pallas as pl
from jax.experimental.pallas import tpu as pltpu
from jax.experimental.pallas import tpu_sc as plsc

LANES = 16
NC, NS = 2, 16
NW = NC * NS
CHUNK = 128


def kernel(head, tail, emb, W, b):
    B = head.shape[0]
    V, D = emb.shape
    bpw = B // NW
    n_chunks = bpw // CHUNK

    def bf16_rtne(x):
        return (x + 0x7FFF + ((x >> 16) & 1)) >> 16

    c0 = lax.bitcast_convert_type(emb[:, 0], jnp.uint32)
    c1 = lax.bitcast_convert_type(emb[:, 1], jnp.uint32)
    packed = ((bf16_rtne(c0) << 16) | bf16_rtne(c1)).astype(jnp.int32)

    head3 = head.reshape(NW, n_chunks, CHUNK)
    tail3 = tail.reshape(NW, n_chunks, CHUNK)
    wb = jnp.concatenate([W.reshape(4), b, jnp.zeros((3,), jnp.float32)])
    wb16 = jnp.broadcast_to(wb[:, None], (8, LANES))

    @functools.partial(
        pl.kernel,
        out_type=jax.ShapeDtypeStruct((B,), jnp.float32),
        mesh=plsc.VectorSubcoreMesh(core_axis_name="c", subcore_axis_name="s"),
        compiler_params=pltpu.CompilerParams(needs_layout_passes=False),
        scratch_types=[
            pltpu.VMEM((n_chunks, CHUNK), jnp.int32),
            pltpu.VMEM((n_chunks, CHUNK), jnp.int32),
            pltpu.VMEM((bpw,), jnp.int32),
            pltpu.VMEM((bpw,), jnp.int32),
            pltpu.VMEM((bpw,), jnp.float32),
            pltpu.VMEM((8, LANES), jnp.float32),
            pltpu.SemaphoreType.DMA,
        ],
    )
    def retrofit(head_h, tail_h, packed_h, wb_h, out_h,
                 hidx, tidx, hw, tw, outv, wbv, sem):
        wid = lax.axis_index("s") * NC + lax.axis_index("c")
        pltpu.sync_copy(head_h.at[wid], hidx)
        pltpu.sync_copy(tail_h.at[wid], tidx)
        pltpu.sync_copy(wb_h, wbv)
        copies = []
        for c in range(n_chunks):
            dst = pl.ds(c * CHUNK, CHUNK)
            copies.append(pltpu.async_copy(packed_h.at[hidx.at[c]], hw.at[dst], sem))
            copies.append(pltpu.async_copy(packed_h.at[tidx.at[c]], tw.at[dst], sem))
        for cp in copies:
            cp.wait()
        w0, w1, w2, w3, bb = wbv[0], wbv[1], wbv[2], wbv[3], wbv[4]
        himask = jnp.full((LANES,), jnp.int32(-65536))

        def unpack2(g):
            hi = plsc.bitcast(g & himask, jnp.float32)
            lo = plsc.bitcast(g << 16, jnp.float32)
            return hi, lo

        for k in range(bpw // LANES):
            sl = pl.ds(k * LANES, LANES)
            h0, h1 = unpack2(hw[sl])
            t0, t1 = unpack2(tw[sl])
            outv[sl] = h0 * w0 + h1 * w1 + t0 * w2 + t1 * w3 + bb
        pltpu.sync_copy(outv, out_h.at[pl.ds(wid * bpw, bpw)])

    return retrofit(head3, tail3, packed, wb16)
